# baseline (speedup 1.0000x reference)
import functools

import jax
import jax.numpy as jnp
from jax import lax
from jax.experimental import pallas as pl
from jax.experimental.pallas import tpu as pltpu
from jax.experimental.pallas import tpu_sc as plsc

N_NODES = 10000
N_EDGES = 320000
D = 128
BM = 1000
K = 40
SCH = 50
ZCHUNKS = N_NODES // K


_GDN = lax.GatherDimensionNumbers(
    offset_dims=(), collapsed_slice_dims=(0,), start_index_map=(0,))


def _shuffle(v, idx):
    return lax.gather(v, idx[:, None], dimension_numbers=_GDN,
                      slice_sizes=(1,),
                      mode=lax.GatherScatterMode.PROMISE_IN_BOUNDS)


def _proj_body(x_ref, wl_ref, wr_ref, w_ref, hl_ref, hrc_ref):
    x = x_ref[...]
    hl_ref[...] = jnp.dot(x, wl_ref[...], preferred_element_type=jnp.float32)
    hrc_ref[:, :D] = jnp.dot(
        x, wr_ref[...], preferred_element_type=jnp.float32).astype(jnp.bfloat16)
    hrc_ref[:, D:] = jnp.dot(
        x, w_ref[...], preferred_element_type=jnp.float32).astype(jnp.bfloat16)


def _project(X, W_l, W_r, W):
    return pl.pallas_call(
        _proj_body,
        grid=(N_NODES // BM,),
        in_specs=[
            pl.BlockSpec((BM, D), lambda i: (i, 0)),
            pl.BlockSpec((D, D), lambda i: (0, 0)),
            pl.BlockSpec((D, D), lambda i: (0, 0)),
            pl.BlockSpec((D, D), lambda i: (0, 0)),
        ],
        out_specs=[
            pl.BlockSpec((BM, D), lambda i: (i, 0)),
            pl.BlockSpec((BM, 2 * D), lambda i: (i, 0)),
        ],
        out_shape=[
            jax.ShapeDtypeStruct((N_NODES, D), jnp.float32),
            jax.ShapeDtypeStruct((N_NODES, 2 * D), jnp.bfloat16),
        ],
    )(X, W_l, W_r, W)


def _sc_gat(HL, HRC, row, col):
    HRC = lax.bitcast_convert_type(
        HRC.reshape(N_NODES, D, 2), jnp.int32)
    info = plsc.get_sparse_core_info()
    NC, NS = info.num_cores, info.num_subcores
    NW = NC * NS
    nsup = N_EDGES // (NW * SCH * K)
    row4 = row.reshape(NW, nsup, SCH, K)
    col4 = col.reshape(NW, nsup, SCH, K)
    mesh = plsc.VectorSubcoreMesh(core_axis_name="c", subcore_axis_name="s")

    @functools.partial(
        pl.kernel, mesh=mesh,
        out_type=jax.ShapeDtypeStruct((NC, N_NODES, D), jnp.float32),
        scratch_types=[
            pltpu.VMEM((SCH, K), jnp.int32),
            pltpu.VMEM((SCH, K), jnp.int32),
            pltpu.VMEM((K, D), jnp.float32),
            pltpu.VMEM((K, D), jnp.int32),
            pltpu.VMEM((K, D), jnp.float32),
            pltpu.VMEM((K, D), jnp.int32),
            pltpu.VMEM((K, D), jnp.float32),
            pltpu.VMEM_SHARED((N_NODES, D), jnp.float32),
            pltpu.SemaphoreType.DMA,
            pltpu.SemaphoreType.DMA,
        ],
    )
    def k(hl_hbm, hrc_hbm, row_hbm, col_hbm, out_hbm,
          rbuf, cbuf, hlv0, hrcv0, hlv1, hrcv1, msgv0,
          agg, gsem0, gsem1):
        hlv = (hlv0, hlv1)
        hrcv = (hrcv0, hrcv1)
        gsem = (gsem0, gsem1)
        cid = lax.axis_index("c")
        sid = lax.axis_index("s")
        wid = sid * NC + cid

        def zrow(kk, carry):
            for g in range(8):
                msgv0[kk, pl.ds(g * 16, 16)] = jnp.zeros((16,), jnp.float32)
            return carry
        lax.fori_loop(0, K, zrow, 0)
        for j in range(ZCHUNKS // NS + 1):
            c = sid + NS * j

            @pl.when(c < ZCHUNKS)
            def _():
                pltpu.sync_copy(msgv0, agg.at[pl.ds(c * K, K)])
        plsc.subcore_barrier()

        def start_gather(t, jj):
            pltpu.async_copy(hl_hbm.at[rbuf.at[jj]], hlv[t], gsem[t])
            pltpu.async_copy(hrc_hbm.at[cbuf.at[jj]], hrcv[t], gsem[t])

        def wait_gather(t):
            pltpu.make_async_copy(hl_hbm.at[rbuf.at[0]], hlv[t],
                                  gsem[t]).wait()
            pltpu.make_async_copy(hrc_hbm.at[cbuf.at[0]], hrcv[t],
                                  gsem[t]).wait()

        def scatter(jj):
            pltpu.sync_copy(msgv0, agg.at[rbuf.at[jj]], add=True)

        lane = jnp.arange(16, dtype=jnp.int32)

        def unpk(x):
            even = lax.bitcast_convert_type(lax.shift_left(x, 16),
                                            jnp.float32)
            odd = lax.bitcast_convert_type(x & jnp.int32(-65536), jnp.float32)
            return even, odd

        def compute(t, jj):
            hl, hrc, msg = hlv[t], hrcv[t], msgv0

            @plsc.parallel_loop(0, K, unroll=2)
            def edge(kk):
                acc = jnp.zeros((16,), jnp.float32)
                for g in range(4):
                    b0, b1 = unpk(hrc[kk, pl.ds(g * 16, 16)])
                    acc = (acc + hl[kk, pl.ds(g * 32, 16)] * b0
                           + hl[kk, pl.ds(g * 32 + 16, 16)] * b1)
                for sh in (1, 2, 4, 8):
                    acc = acc + _shuffle(acc, lane ^ sh)
                e = jnp.where(acc >= 0.0, acc, acc * jnp.float32(0.2))
                for g in range(4):
                    c0, c1 = unpk(hrc[kk, pl.ds(D // 2 + g * 16, 16)])
                    msg[kk, pl.ds(g * 32, 16)] = e * c0
                    msg[kk, pl.ds(g * 32 + 16, 16)] = e * c1

        def pair(p, carry):
            start_gather(1, 2 * p + 1)
            wait_gather(0)
            compute(0, 2 * p)
            scatter(2 * p)
            @pl.when(p < SCH // 2 - 1)
            def _():
                start_gather(0, 2 * p + 2)
            wait_gather(1)
            compute(1, 2 * p + 1)
            scatter(2 * p + 1)
            return carry

        def superchunk(s, carry):
            pltpu.sync_copy(row_hbm.at[wid, s], rbuf)
            pltpu.sync_copy(col_hbm.at[wid, s], cbuf)
            start_gather(0, 0)
            lax.fori_loop(0, SCH // 2, pair, 0)
            return carry
        lax.fori_loop(0, nsup, superchunk, 0)
        plsc.subcore_barrier()

        for j in range(ZCHUNKS // NS + 1):
            c = sid + NS * j

            @pl.when(c < ZCHUNKS)
            def _():
                pltpu.sync_copy(agg.at[pl.ds(c * K, K)],
                                out_hbm.at[cid, pl.ds(c * K, K)])

    return k(HL, HRC, row4, col4)


def _combine_body(p_ref, o_ref):
    o_ref[...] = jnp.sum(p_ref[...], axis=0)


def _combine(parts):
    nc = parts.shape[0]
    return pl.pallas_call(
        _combine_body,
        grid=(N_NODES // BM,),
        in_specs=[pl.BlockSpec((nc, BM, D), lambda i: (0, i, 0))],
        out_specs=pl.BlockSpec((BM, D), lambda i: (i, 0)),
        out_shape=jax.ShapeDtypeStruct((N_NODES, D), jnp.float32),
    )(parts)


import numpy as _np

_q = _np.arange(D) % 32
_g = (_np.arange(D) // 32) * 32
_COMB_NAT = _g + _np.where(_q % 2 == 0, _q // 2, 16 + (_q - 1) // 2)


def kernel(X, edge_index, W, W_r, W_l):
    HL, HRC = _project(X, W_l, jnp.asarray(W_r)[:, _COMB_NAT],
                       jnp.asarray(W)[:, _COMB_NAT])
    row = edge_index[0]
    col = edge_index[1]
    parts = _sc_gat(HL, HRC, row, col)
    return _combine(parts)


# speedup vs baseline: 1.2137x; 1.2137x over previous
import functools

import jax
import jax.numpy as jnp
from jax import lax
from jax.experimental import pallas as pl
from jax.experimental.pallas import tpu as pltpu
from jax.experimental.pallas import tpu_sc as plsc

N_NODES = 10000
N_EDGES = 320000
D = 128
BM = 1000
K = 40
SCH = 50
ZCHUNKS = N_NODES // K


_GDN = lax.GatherDimensionNumbers(
    offset_dims=(), collapsed_slice_dims=(0,), start_index_map=(0,))


def _shuffle(v, idx):
    return lax.gather(v, idx[:, None], dimension_numbers=_GDN,
                      slice_sizes=(1,),
                      mode=lax.GatherScatterMode.PROMISE_IN_BOUNDS)


def _proj_body(x_ref, wl_ref, wr_ref, w_ref, hl_ref, hrc_ref):
    x = x_ref[...]
    hl_ref[...] = jnp.dot(x, wl_ref[...], preferred_element_type=jnp.float32)
    hrc_ref[:, :D] = jnp.dot(x, wr_ref[...], preferred_element_type=jnp.float32)
    hrc_ref[:, D:] = jnp.dot(x, w_ref[...], preferred_element_type=jnp.float32)


def _project(X, W_l, W_r, W):
    return pl.pallas_call(
        _proj_body,
        grid=(N_NODES // BM,),
        in_specs=[
            pl.BlockSpec((BM, D), lambda i: (i, 0)),
            pl.BlockSpec((D, D), lambda i: (0, 0)),
            pl.BlockSpec((D, D), lambda i: (0, 0)),
            pl.BlockSpec((D, D), lambda i: (0, 0)),
        ],
        out_specs=[
            pl.BlockSpec((BM, D), lambda i: (i, 0)),
            pl.BlockSpec((BM, 2 * D), lambda i: (i, 0)),
        ],
        out_shape=[
            jax.ShapeDtypeStruct((N_NODES, D), jnp.float32),
            jax.ShapeDtypeStruct((N_NODES, 2 * D), jnp.float32),
        ],
    )(X, W_l, W_r, W)


def _sc_gat(HL, HRC, row, col):
    info = plsc.get_sparse_core_info()
    NC, NS = info.num_cores, info.num_subcores
    NW = NC * NS
    nsup = N_EDGES // (NW * SCH * K)
    row4 = row.reshape(NW, nsup, SCH, K)
    col4 = col.reshape(NW, nsup, SCH, K)
    mesh = plsc.VectorSubcoreMesh(core_axis_name="c", subcore_axis_name="s")

    @functools.partial(
        pl.kernel, mesh=mesh,
        out_type=jax.ShapeDtypeStruct((NC, N_NODES, D), jnp.float32),
        scratch_types=[
            pltpu.VMEM((SCH, K), jnp.int32),
            pltpu.VMEM((SCH, K), jnp.int32),
            pltpu.VMEM((K, D), jnp.float32),
            pltpu.VMEM((K, 2 * D), jnp.float32),
            pltpu.VMEM((K, D), jnp.float32),
            pltpu.VMEM((K, 2 * D), jnp.float32),
            pltpu.VMEM((K, D), jnp.float32),
            pltpu.VMEM_SHARED((N_NODES, D), jnp.float32),
            pltpu.SemaphoreType.DMA,
            pltpu.SemaphoreType.DMA,
            pltpu.SemaphoreType.DMA,
        ],
    )
    def k(hl_hbm, hrc_hbm, row_hbm, col_hbm, out_hbm,
          rbuf, cbuf, hlv0, hrcv0, hlv1, hrcv1, msgv0,
          agg, gsem0, gsem1, ssem):
        hlv = (hlv0, hlv1)
        hrcv = (hrcv0, hrcv1)
        gsem = (gsem0, gsem1)
        cid = lax.axis_index("c")
        sid = lax.axis_index("s")
        wid = sid * NC + cid

        def zrow(kk, carry):
            for g in range(8):
                msgv0[kk, pl.ds(g * 16, 16)] = jnp.zeros((16,), jnp.float32)
            return carry
        lax.fori_loop(0, K, zrow, 0)
        for j in range(ZCHUNKS // NS + 1):
            c = sid + NS * j

            @pl.when(c < ZCHUNKS)
            def _():
                pltpu.sync_copy(msgv0, agg.at[pl.ds(c * K, K)])
        plsc.subcore_barrier()

        def start_gather(t, jj):
            pltpu.async_copy(hl_hbm.at[rbuf.at[jj]], hlv[t], gsem[t])
            pltpu.async_copy(hrc_hbm.at[cbuf.at[jj]], hrcv[t], gsem[t])

        def wait_gather(t):
            pltpu.make_async_copy(hl_hbm.at[rbuf.at[0]], hlv[t],
                                  gsem[t]).wait()
            pltpu.make_async_copy(hrc_hbm.at[cbuf.at[0]], hrcv[t],
                                  gsem[t]).wait()

        def start_scatter(jj):
            pltpu.async_copy(msgv0, agg.at[rbuf.at[jj]], ssem, add=True)

        def wait_scatter():
            pltpu.make_async_copy(msgv0, agg.at[rbuf.at[0]], ssem).wait()

        lane = jnp.arange(16, dtype=jnp.int32)

        def unpk(x):
            even = lax.bitcast_convert_type(lax.shift_left(x, 16),
                                            jnp.float32)
            odd = lax.bitcast_convert_type(x & jnp.int32(-65536), jnp.float32)
            return even, odd

        def compute(t, jj):
            hl, hrc, msg = hlv[t], hrcv[t], msgv0

            @plsc.parallel_loop(0, K, unroll=2)
            def edge(kk):
                acc = jnp.zeros((16,), jnp.float32)
                for g in range(8):
                    acc = acc + (hl[kk, pl.ds(g * 16, 16)]
                                 * hrc[kk, pl.ds(g * 16, 16)])
                for sh in (1, 2, 4, 8):
                    acc = acc + _shuffle(acc, lane ^ sh)
                e = jnp.where(acc >= 0.0, acc, acc * jnp.float32(0.2))
                for g in range(8):
                    msg[kk, pl.ds(g * 16, 16)] = (
                        e * hrc[kk, pl.ds(D + g * 16, 16)])

        def pair(p, carry):
            start_gather(1, 2 * p + 1)
            wait_gather(0)
            @pl.when(carry >= 1)
            def _():
                wait_scatter()
            compute(0, 2 * p)
            start_scatter(2 * p)
            @pl.when(p < SCH // 2 - 1)
            def _():
                start_gather(0, 2 * p + 2)
            wait_gather(1)
            wait_scatter()
            compute(1, 2 * p + 1)
            start_scatter(2 * p + 1)
            return jnp.int32(1)

        def superchunk(s, carry):
            @pl.when(s >= 1)
            def _():
                wait_scatter()
            pltpu.sync_copy(row_hbm.at[wid, s], rbuf)
            pltpu.sync_copy(col_hbm.at[wid, s], cbuf)
            start_gather(0, 0)
            lax.fori_loop(0, SCH // 2, pair, 0)
            return carry
        lax.fori_loop(0, nsup, superchunk, 0)
        wait_scatter()
        plsc.subcore_barrier()

        for j in range(ZCHUNKS // NS + 1):
            c = sid + NS * j

            @pl.when(c < ZCHUNKS)
            def _():
                pltpu.sync_copy(agg.at[pl.ds(c * K, K)],
                                out_hbm.at[cid, pl.ds(c * K, K)])

    return k(HL, HRC, row4, col4)


def _combine_body(p_ref, o_ref):
    o_ref[...] = jnp.sum(p_ref[...], axis=0)


def _combine(parts):
    nc = parts.shape[0]
    return pl.pallas_call(
        _combine_body,
        grid=(N_NODES // BM,),
        in_specs=[pl.BlockSpec((nc, BM, D), lambda i: (0, i, 0))],
        out_specs=pl.BlockSpec((BM, D), lambda i: (i, 0)),
        out_shape=jax.ShapeDtypeStruct((N_NODES, D), jnp.float32),
    )(parts)


import numpy as _np

_q = _np.arange(D) % 32
_g = (_np.arange(D) // 32) * 32
_COMB_NAT = _g + _np.where(_q % 2 == 0, _q // 2, 16 + (_q - 1) // 2)


def kernel(X, edge_index, W, W_r, W_l):
    HL, HRC = _project(X, W_l, W_r, W)
    row = edge_index[0]
    col = edge_index[1]
    parts = _sc_gat(HL, HRC, row, col)
    return _combine(parts)
